# baseline (speedup 1.0000x reference)
import jax
import jax.numpy as jnp
from jax.experimental import pallas as pl
from jax.experimental.pallas import tpu as pltpu

N_TOKENS = 32768
D_MODEL = 768
N_EXPERTS = 8
TEMPERATURE = 1.0

BLK = 2048


def _router_block(x_ref, wt_ref, out_ref):
    xb = x_ref[...].astype(jnp.bfloat16)
    out_ref[...] = jnp.dot(xb, wt_ref[...], preferred_element_type=jnp.float32)


def kernel(x, W):
    n_tokens, d_model = x.shape
    n_experts = W.shape[0]
    wt = W.T.astype(jnp.bfloat16)

    grid = (n_tokens // BLK,)
    logits = pl.pallas_call(
        _router_block,
        grid=grid,
        in_specs=[
            pl.BlockSpec((BLK, d_model), lambda i: (i, 0)),
            pl.BlockSpec((d_model, n_experts), lambda i: (0, 0)),
        ],
        out_specs=pl.BlockSpec((BLK, n_experts), lambda i: (i, 0)),
        out_shape=jax.ShapeDtypeStruct((n_tokens, n_experts), jnp.float32),
        compiler_params=pltpu.CompilerParams(
            dimension_semantics=("parallel",),
        ),
    )(x, wt)

    temp = max(TEMPERATURE, 1e-06)
    if temp != 1.0:
        logits = logits / temp
    return logits


# speedup vs baseline: 1.0046x; 1.0046x over previous
import jax
import jax.numpy as jnp
from jax.experimental import pallas as pl
from jax.experimental.pallas import tpu as pltpu

N_TOKENS = 32768
D_MODEL = 768
N_EXPERTS = 8
TEMPERATURE = 1.0

CH = 1024
NB = 3
NSCR = 4
HALF = N_TOKENS // 2


def _router_kernel(xa, xb, wt_ref, out_hbm, bufa, bufb, outbuf, insems_a, insems_b, outsems):
    n_chunks = HALF // CH
    wt = wt_ref[...]

    def copy_in(ref, c, base, bufs, sems, buf, prio):
        pltpu.async_copy(
            ref.at[pl.ds(base + c * CH, CH), :], bufs.at[buf], sems.at[buf],
            priority=prio,
        )

    for c in range(min(NB, n_chunks)):
        copy_in(xa, c, 0, bufa, insems_a, c, 0)
        copy_in(xb, c, HALF, bufb, insems_b, c, 1)

    outs_started = [False] * NSCR
    step = 0
    for c in range(n_chunks):
        for s in range(2):
            ref = xa if s == 0 else xb
            bufs = bufa if s == 0 else bufb
            sems = insems_a if s == 0 else insems_b
            base = 0 if s == 0 else HALF
            buf = c % NB
            pltpu.make_async_copy(
                ref.at[pl.ds(base + c * CH, CH), :], bufs.at[buf], sems.at[buf]
            ).wait()
            slot = step % NSCR
            if outs_started[slot]:
                prev = step - NSCR
                pbase = 0 if prev % 2 == 0 else HALF
                prow = (prev // 2) * CH + pbase
                pltpu.make_async_copy(
                    outbuf.at[slot], out_hbm.at[pl.ds(prow, CH), :], outsems.at[slot]
                ).wait()
            xc = bufs[buf].astype(jnp.bfloat16)
            outbuf[slot] = jnp.dot(xc, wt, preferred_element_type=jnp.float32)
            row = base + c * CH
            pltpu.make_async_copy(
                outbuf.at[slot], out_hbm.at[pl.ds(row, CH), :], outsems.at[slot]
            ).start()
            outs_started[slot] = True
            nxt = c + NB
            if nxt < n_chunks:
                copy_in(ref, nxt, base, bufs, sems, buf, s)
            step += 1

    total = 2 * n_chunks
    for back in range(min(NSCR, total)):
        prev = total - 1 - back
        slot = prev % NSCR
        pbase = 0 if prev % 2 == 0 else HALF
        prow = (prev // 2) * CH + pbase
        pltpu.make_async_copy(
            outbuf.at[slot], out_hbm.at[pl.ds(prow, CH), :], outsems.at[slot]
        ).wait()


def kernel(x, W):
    n_tokens, d_model = x.shape
    n_experts = W.shape[0]
    wt = W.T.astype(jnp.bfloat16)

    logits = pl.pallas_call(
        _router_kernel,
        in_specs=[
            pl.BlockSpec(memory_space=pltpu.MemorySpace.HBM),
            pl.BlockSpec(memory_space=pltpu.MemorySpace.HBM),
            pl.BlockSpec(memory_space=pltpu.MemorySpace.VMEM),
        ],
        out_specs=pl.BlockSpec(memory_space=pltpu.MemorySpace.HBM),
        out_shape=jax.ShapeDtypeStruct((n_tokens, n_experts), jnp.float32),
        scratch_shapes=[
            pltpu.VMEM((NB, CH, D_MODEL), jnp.float32),
            pltpu.VMEM((NB, CH, D_MODEL), jnp.float32),
            pltpu.VMEM((NSCR, CH, N_EXPERTS), jnp.float32),
            pltpu.SemaphoreType.DMA((NB,)),
            pltpu.SemaphoreType.DMA((NB,)),
            pltpu.SemaphoreType.DMA((NSCR,)),
        ],
    )(x, x, wt)

    temp = max(TEMPERATURE, 1e-06)
    if temp != 1.0:
        logits = logits / temp
    return logits


# speedup vs baseline: 1.0566x; 1.0518x over previous
import jax
import jax.numpy as jnp
from jax import lax
from jax.experimental import pallas as pl

N_TOKENS = 32768
D_MODEL = 768
N_EXPERTS = 8
TEMPERATURE = 1.0

BLK = 4096


def _router_block(x_ref, w_ref, out_ref):
    xb = x_ref[...].astype(jnp.bfloat16)
    wb = w_ref[...].astype(jnp.bfloat16)
    out_ref[...] = lax.dot_general(
        xb, wb, (((1,), (1,)), ((), ())), preferred_element_type=jnp.float32
    )


def kernel(x, W):
    n_tokens, d_model = x.shape
    n_experts = W.shape[0]

    grid = (n_tokens // BLK,)
    logits = pl.pallas_call(
        _router_block,
        grid=grid,
        in_specs=[
            pl.BlockSpec((BLK, d_model), lambda i: (i, 0)),
            pl.BlockSpec((n_experts, d_model), lambda i: (0, 0)),
        ],
        out_specs=pl.BlockSpec((BLK, n_experts), lambda i: (i, 0)),
        out_shape=jax.ShapeDtypeStruct((n_tokens, n_experts), jnp.float32),
    )(x, W)

    temp = max(TEMPERATURE, 1e-06)
    if temp != 1.0:
        logits = logits / temp
    return logits
